# baseline (speedup 1.0000x reference)
import functools

import jax
import jax.numpy as jnp
from jax.experimental import pallas as pl
from jax.experimental.pallas import tpu as pltpu


_BT = 2048
_NBUF = 4

_K0 = 0
_K1 = 42
_KS2 = 0x1BD11BDA ^ _K0 ^ _K1
_ROT_A = (13, 15, 26, 6)
_ROT_B = (17, 29, 16, 24)


def _rotl(v, d):
    return (v << jnp.uint32(d)) | (v >> jnp.uint32(32 - d))


def _threefry_bits(cnt):
    ks0 = jnp.uint32(_K0)
    ks1 = jnp.uint32(_K1)
    ks2 = jnp.uint32(_KS2)
    x0 = jnp.zeros_like(cnt) + ks0
    x1 = cnt + ks1

    def four(x0, x1, rots):
        for r in rots:
            x0 = x0 + x1
            x1 = _rotl(x1, r)
            x1 = x1 ^ x0
        return x0, x1

    x0, x1 = four(x0, x1, _ROT_A)
    x0 = x0 + ks1
    x1 = x1 + ks2 + jnp.uint32(1)
    x0, x1 = four(x0, x1, _ROT_B)
    x0 = x0 + ks2
    x1 = x1 + ks0 + jnp.uint32(2)
    x0, x1 = four(x0, x1, _ROT_A)
    x0 = x0 + ks0
    x1 = x1 + ks1 + jnp.uint32(3)
    x0, x1 = four(x0, x1, _ROT_B)
    x0 = x0 + ks1
    x1 = x1 + ks2 + jnp.uint32(4)
    x0, x1 = four(x0, x1, _ROT_A)
    x0 = x0 + ks2
    x1 = x1 + ks0 + jnp.uint32(5)
    return x0 ^ x1


def _gumbel_block(t0, n_experts, bt):
    e_iota = jax.lax.broadcasted_iota(jnp.uint32, (n_experts, bt), 0)
    t_iota = jax.lax.broadcasted_iota(jnp.uint32, (n_experts, bt), 1)
    cnt = (jnp.uint32(t0) + t_iota) * jnp.uint32(n_experts) + e_iota
    bits = _threefry_bits(cnt)
    fbits = (bits >> jnp.uint32(9)) | jnp.uint32(0x3F800000)
    flo = jax.lax.bitcast_convert_type(fbits, jnp.float32) - jnp.float32(1.0)
    minval = jnp.float32(1e-20)
    maxval = jnp.float32(1.0)
    u = jnp.maximum(minval, flo * (maxval - minval) + minval)
    g = -jnp.log(-jnp.log(u))
    return g.T


def _gate_kernel(x_hbm, wt_ref, b_ref, o_ref, xbuf, sems):
    i = pl.program_id(0)
    nsteps = pl.num_programs(0)
    n_experts = o_ref.shape[-1]

    def copy(step, slot):
        return pltpu.make_async_copy(
            x_hbm.at[pl.ds(step * _BT, _BT), :],
            xbuf.at[slot],
            sems.at[slot])

    @pl.when(i == 0)
    def _warmup():
        for k in range(_NBUF - 1):
            copy(k, k).start()

    nxt = i + _NBUF - 1

    @pl.when(nxt < nsteps)
    def _prefetch():
        copy(nxt, jax.lax.rem(nxt, _NBUF)).start()

    gumbels = _gumbel_block(i * _BT, n_experts, _BT)

    slot = jax.lax.rem(i, _NBUF)
    copy(i, slot).wait()
    xblk = xbuf[slot]

    scores = jnp.dot(xblk, wt_ref[...], preferred_element_type=jnp.float32)
    y = scores + b_ref[...] + gumbels
    idx = jnp.argmax(y, axis=-1)
    expert = jax.lax.broadcasted_iota(jnp.int32, y.shape, 1)
    o_ref[...] = (expert == idx[:, None]).astype(jnp.float32)


@functools.partial(jax.jit, static_argnames=())
def kernel(x, gate_weight, gate_bias):
    n_tokens, d_model = x.shape
    n_experts = gate_weight.shape[0]
    wt = gate_weight.T
    b2 = gate_bias.reshape(1, n_experts)
    grid = (n_tokens // _BT,)
    return pl.pallas_call(
        _gate_kernel,
        grid=grid,
        in_specs=[
            pl.BlockSpec(memory_space=pl.ANY),
            pl.BlockSpec((d_model, n_experts), lambda i: (0, 0)),
            pl.BlockSpec((1, n_experts), lambda i: (0, 0)),
        ],
        out_specs=pl.BlockSpec((_BT, n_experts), lambda i: (i, 0)),
        out_shape=jax.ShapeDtypeStruct((n_tokens, n_experts), x.dtype),
        scratch_shapes=[
            pltpu.VMEM((_NBUF, _BT, d_model), jnp.float32),
            pltpu.SemaphoreType.DMA((_NBUF,)),
        ],
    )(x, wt, b2)


# speedup vs baseline: 1.0094x; 1.0094x over previous
import functools

import jax
import jax.numpy as jnp
from jax.experimental import pallas as pl
from jax.experimental.pallas import tpu as pltpu


_BT = 1024
_NBUF = 8

_K0 = 0
_K1 = 42
_KS2 = 0x1BD11BDA ^ _K0 ^ _K1
_ROT_A = (13, 15, 26, 6)
_ROT_B = (17, 29, 16, 24)


def _rotl(v, d):
    return (v << jnp.uint32(d)) | (v >> jnp.uint32(32 - d))


def _threefry_bits(cnt):
    ks0 = jnp.uint32(_K0)
    ks1 = jnp.uint32(_K1)
    ks2 = jnp.uint32(_KS2)
    x0 = jnp.zeros_like(cnt) + ks0
    x1 = cnt + ks1

    def four(x0, x1, rots):
        for r in rots:
            x0 = x0 + x1
            x1 = _rotl(x1, r)
            x1 = x1 ^ x0
        return x0, x1

    x0, x1 = four(x0, x1, _ROT_A)
    x0 = x0 + ks1
    x1 = x1 + ks2 + jnp.uint32(1)
    x0, x1 = four(x0, x1, _ROT_B)
    x0 = x0 + ks2
    x1 = x1 + ks0 + jnp.uint32(2)
    x0, x1 = four(x0, x1, _ROT_A)
    x0 = x0 + ks0
    x1 = x1 + ks1 + jnp.uint32(3)
    x0, x1 = four(x0, x1, _ROT_B)
    x0 = x0 + ks1
    x1 = x1 + ks2 + jnp.uint32(4)
    x0, x1 = four(x0, x1, _ROT_A)
    x0 = x0 + ks2
    x1 = x1 + ks0 + jnp.uint32(5)
    return x0 ^ x1


def _gumbel_block(t0, n_experts, bt):
    e_iota = jax.lax.broadcasted_iota(jnp.uint32, (n_experts, bt), 0)
    t_iota = jax.lax.broadcasted_iota(jnp.uint32, (n_experts, bt), 1)
    cnt = (jnp.uint32(t0) + t_iota) * jnp.uint32(n_experts) + e_iota
    bits = _threefry_bits(cnt)
    fbits = (bits >> jnp.uint32(9)) | jnp.uint32(0x3F800000)
    flo = jax.lax.bitcast_convert_type(fbits, jnp.float32) - jnp.float32(1.0)
    minval = jnp.float32(1e-20)
    maxval = jnp.float32(1.0)
    u = jnp.maximum(minval, flo * (maxval - minval) + minval)
    g = -jnp.log(-jnp.log(u))
    return g.T


def _gate_kernel(x_hbm, wt_ref, b_ref, o_ref, xbuf, sems):
    i = pl.program_id(0)
    nsteps = pl.num_programs(0)
    n_experts = o_ref.shape[-1]

    def copy(step, slot):
        return pltpu.make_async_copy(
            x_hbm.at[pl.ds(step * _BT, _BT), :],
            xbuf.at[slot],
            sems.at[slot])

    @pl.when(i == 0)
    def _warmup():
        for k in range(_NBUF - 1):
            copy(k, k).start()

    nxt = i + _NBUF - 1

    @pl.when(nxt < nsteps)
    def _prefetch():
        copy(nxt, jax.lax.rem(nxt, _NBUF)).start()

    gumbels = _gumbel_block(i * _BT, n_experts, _BT)

    slot = jax.lax.rem(i, _NBUF)
    copy(i, slot).wait()
    xblk = xbuf[slot]

    scores = jnp.dot(xblk, wt_ref[...], preferred_element_type=jnp.float32)
    y = scores + b_ref[...] + gumbels
    idx = jnp.argmax(y, axis=-1)
    expert = jax.lax.broadcasted_iota(jnp.int32, y.shape, 1)
    o_ref[...] = (expert == idx[:, None]).astype(jnp.float32)


@functools.partial(jax.jit, static_argnames=())
def kernel(x, gate_weight, gate_bias):
    n_tokens, d_model = x.shape
    n_experts = gate_weight.shape[0]
    wt = gate_weight.T
    b2 = gate_bias.reshape(1, n_experts)
    grid = (n_tokens // _BT,)
    return pl.pallas_call(
        _gate_kernel,
        grid=grid,
        in_specs=[
            pl.BlockSpec(memory_space=pl.ANY),
            pl.BlockSpec((d_model, n_experts), lambda i: (0, 0)),
            pl.BlockSpec((1, n_experts), lambda i: (0, 0)),
        ],
        out_specs=pl.BlockSpec((_BT, n_experts), lambda i: (i, 0)),
        out_shape=jax.ShapeDtypeStruct((n_tokens, n_experts), x.dtype),
        scratch_shapes=[
            pltpu.VMEM((_NBUF, _BT, d_model), jnp.float32),
            pltpu.SemaphoreType.DMA((_NBUF,)),
        ],
    )(x, wt, b2)


# speedup vs baseline: 1.0557x; 1.0459x over previous
import functools

import jax
import jax.numpy as jnp
from jax.experimental import pallas as pl
from jax.experimental.pallas import tpu as pltpu


_BT = 2048
_NBUF = 4

_K0 = 0
_K1 = 42
_KS2 = 0x1BD11BDA ^ _K0 ^ _K1
_ROT_A = (13, 15, 26, 6)
_ROT_B = (17, 29, 16, 24)


def _rotl(v, d):
    return (v << jnp.uint32(d)) | (v >> jnp.uint32(32 - d))


def _threefry_bits(cnt):
    ks0 = jnp.uint32(_K0)
    ks1 = jnp.uint32(_K1)
    ks2 = jnp.uint32(_KS2)
    x0 = jnp.zeros_like(cnt) + ks0
    x1 = cnt + ks1

    def four(x0, x1, rots):
        for r in rots:
            x0 = x0 + x1
            x1 = _rotl(x1, r)
            x1 = x1 ^ x0
        return x0, x1

    x0, x1 = four(x0, x1, _ROT_A)
    x0 = x0 + ks1
    x1 = x1 + ks2 + jnp.uint32(1)
    x0, x1 = four(x0, x1, _ROT_B)
    x0 = x0 + ks2
    x1 = x1 + ks0 + jnp.uint32(2)
    x0, x1 = four(x0, x1, _ROT_A)
    x0 = x0 + ks0
    x1 = x1 + ks1 + jnp.uint32(3)
    x0, x1 = four(x0, x1, _ROT_B)
    x0 = x0 + ks1
    x1 = x1 + ks2 + jnp.uint32(4)
    x0, x1 = four(x0, x1, _ROT_A)
    x0 = x0 + ks2
    x1 = x1 + ks0 + jnp.uint32(5)
    return x0 ^ x1


def _gumbel_block(t0, n_experts, bt):
    e_iota = jax.lax.broadcasted_iota(jnp.uint32, (n_experts, bt), 0)
    t_iota = jax.lax.broadcasted_iota(jnp.uint32, (n_experts, bt), 1)
    cnt = (jnp.uint32(t0) + t_iota) * jnp.uint32(n_experts) + e_iota
    bits = _threefry_bits(cnt)
    fbits = (bits >> jnp.uint32(9)) | jnp.uint32(0x3F800000)
    flo = jax.lax.bitcast_convert_type(fbits, jnp.float32) - jnp.float32(1.0)
    minval = jnp.float32(1e-20)
    maxval = jnp.float32(1.0)
    u = jnp.maximum(minval, flo * (maxval - minval) + minval)
    g = -jnp.log(-jnp.log(u))
    return g.T


def _gate_kernel(x_hbm, wt_ref, b_ref, o_ref, xbuf, sems):
    i = pl.program_id(0)
    nsteps = pl.num_programs(0)
    n_experts = o_ref.shape[-1]

    def copy(step, slot):
        return pltpu.make_async_copy(
            x_hbm.at[pl.ds(step * _BT, _BT), :],
            xbuf.at[slot],
            sems.at[slot])

    @pl.when(i == 0)
    def _warmup():
        for k in range(_NBUF - 1):
            copy(k, k).start()

    nxt = i + _NBUF - 1

    @pl.when(nxt < nsteps)
    def _prefetch():
        copy(nxt, jax.lax.rem(nxt, _NBUF)).start()

    gumbels = _gumbel_block(i * _BT, n_experts, _BT)

    slot = jax.lax.rem(i, _NBUF)
    copy(i, slot).wait()
    xblk = xbuf[slot]

    scores = jax.lax.dot_general(
        xblk, wt_ref[...], (((1,), (1,)), ((), ())),
        preferred_element_type=jnp.float32)
    y = scores + b_ref[...] + gumbels
    idx = jnp.argmax(y, axis=-1)
    expert = jax.lax.broadcasted_iota(jnp.int32, y.shape, 1)
    o_ref[...] = (expert == idx[:, None]).astype(jnp.float32)


@functools.partial(jax.jit, static_argnames=())
def kernel(x, gate_weight, gate_bias):
    n_tokens, d_model = x.shape
    n_experts = gate_weight.shape[0]
    b2 = gate_bias.reshape(1, n_experts)
    grid = (n_tokens // _BT,)
    return pl.pallas_call(
        _gate_kernel,
        grid=grid,
        in_specs=[
            pl.BlockSpec(memory_space=pl.ANY),
            pl.BlockSpec((n_experts, d_model), lambda i: (0, 0)),
            pl.BlockSpec((1, n_experts), lambda i: (0, 0)),
        ],
        out_specs=pl.BlockSpec((_BT, n_experts), lambda i: (i, 0)),
        out_shape=jax.ShapeDtypeStruct((n_tokens, n_experts), x.dtype),
        scratch_shapes=[
            pltpu.VMEM((_NBUF, _BT, d_model), jnp.float32),
            pltpu.SemaphoreType.DMA((_NBUF,)),
        ],
    )(x, gate_weight, b2)


# speedup vs baseline: 1.4047x; 1.3306x over previous
import functools

import jax
import jax.numpy as jnp
from jax.experimental import pallas as pl
from jax.experimental.pallas import tpu as pltpu


_BT = 4096
_NBUF = 4

_K0 = 0
_K1 = 42
_KS2 = 0x1BD11BDA ^ _K0 ^ _K1
_ROT_A = (13, 15, 26, 6)
_ROT_B = (17, 29, 16, 24)


def _rotl(v, d):
    return (v << jnp.uint32(d)) | (v >> jnp.uint32(32 - d))


def _threefry_bits(cnt):
    ks0 = jnp.uint32(_K0)
    ks1 = jnp.uint32(_K1)
    ks2 = jnp.uint32(_KS2)
    x0 = jnp.zeros_like(cnt) + ks0
    x1 = cnt + ks1

    def four(x0, x1, rots):
        for r in rots:
            x0 = x0 + x1
            x1 = _rotl(x1, r)
            x1 = x1 ^ x0
        return x0, x1

    x0, x1 = four(x0, x1, _ROT_A)
    x0 = x0 + ks1
    x1 = x1 + ks2 + jnp.uint32(1)
    x0, x1 = four(x0, x1, _ROT_B)
    x0 = x0 + ks2
    x1 = x1 + ks0 + jnp.uint32(2)
    x0, x1 = four(x0, x1, _ROT_A)
    x0 = x0 + ks0
    x1 = x1 + ks1 + jnp.uint32(3)
    x0, x1 = four(x0, x1, _ROT_B)
    x0 = x0 + ks1
    x1 = x1 + ks2 + jnp.uint32(4)
    x0, x1 = four(x0, x1, _ROT_A)
    x0 = x0 + ks2
    x1 = x1 + ks0 + jnp.uint32(5)
    return x0 ^ x1


def _gumbel_block(t0, n_experts, bt):
    e_iota = jax.lax.broadcasted_iota(jnp.uint32, (n_experts, bt), 0)
    t_iota = jax.lax.broadcasted_iota(jnp.uint32, (n_experts, bt), 1)
    cnt = (jnp.uint32(t0) + t_iota) * jnp.uint32(n_experts) + e_iota
    bits = _threefry_bits(cnt)
    fbits = (bits >> jnp.uint32(9)) | jnp.uint32(0x3F800000)
    flo = jax.lax.bitcast_convert_type(fbits, jnp.float32) - jnp.float32(1.0)
    minval = jnp.float32(1e-20)
    maxval = jnp.float32(1.0)
    u = jnp.maximum(minval, flo * (maxval - minval) + minval)
    return -jnp.log(-jnp.log(u))


def _gate_kernel(xblk_ref, wt_ref, b_ref, o_ref):
    i = pl.program_id(0)
    n_experts = o_ref.shape[0]

    gumbels = _gumbel_block(i * _BT, n_experts, _BT)
    xblk = xblk_ref[...]

    scores = jax.lax.dot_general(
        xblk, wt_ref[...], (((1,), (1,)), ((), ())),
        preferred_element_type=jnp.float32)
    y_t = scores.T + b_ref[...] + gumbels
    m = jnp.max(y_t, axis=0, keepdims=True)
    e_iota = jax.lax.broadcasted_iota(jnp.int32, y_t.shape, 0)
    sel = jnp.min(jnp.where(y_t == m, e_iota, n_experts), axis=0, keepdims=True)
    o_ref[...] = (e_iota == sel).astype(jnp.float32)


@functools.partial(jax.jit, static_argnames=())
def kernel(x, gate_weight, gate_bias):
    n_tokens, d_model = x.shape
    n_experts = gate_weight.shape[0]
    b2 = gate_bias.reshape(n_experts, 1)
    grid = (n_tokens // _BT,)
    return pl.pallas_call(
        _gate_kernel,
        grid=grid,
        in_specs=[
            pl.BlockSpec((_BT, d_model), lambda i: (i, 0)),
            pl.BlockSpec((n_experts, d_model), lambda i: (0, 0)),
            pl.BlockSpec((n_experts, 1), lambda i: (0, 0)),
        ],
        out_specs=pl.BlockSpec((n_experts, _BT), lambda i: (0, i)),
        out_shape=jax.ShapeDtypeStruct((n_experts, n_tokens), x.dtype),
        compiler_params=pltpu.CompilerParams(
            dimension_semantics=("parallel",)),
    )(x, gate_weight, b2).T


# speedup vs baseline: 1.4114x; 1.0048x over previous
import functools

import jax
import jax.numpy as jnp
from jax.experimental import pallas as pl
from jax.experimental.pallas import tpu as pltpu


_BT = 4096

_K0 = 0
_K1 = 42
_KS2 = 0x1BD11BDA ^ _K0 ^ _K1
_ROT_A = (13, 15, 26, 6)
_ROT_B = (17, 29, 16, 24)


def _rotl(v, d):
    return (v << jnp.uint32(d)) | (v >> jnp.uint32(32 - d))


def _threefry_bits(cnt):
    ks0 = jnp.uint32(_K0)
    ks1 = jnp.uint32(_K1)
    ks2 = jnp.uint32(_KS2)
    x0 = jnp.zeros_like(cnt) + ks0
    x1 = cnt + ks1

    def four(x0, x1, rots):
        for r in rots:
            x0 = x0 + x1
            x1 = _rotl(x1, r)
            x1 = x1 ^ x0
        return x0, x1

    x0, x1 = four(x0, x1, _ROT_A)
    x0 = x0 + ks1
    x1 = x1 + ks2 + jnp.uint32(1)
    x0, x1 = four(x0, x1, _ROT_B)
    x0 = x0 + ks2
    x1 = x1 + ks0 + jnp.uint32(2)
    x0, x1 = four(x0, x1, _ROT_A)
    x0 = x0 + ks0
    x1 = x1 + ks1 + jnp.uint32(3)
    x0, x1 = four(x0, x1, _ROT_B)
    x0 = x0 + ks1
    x1 = x1 + ks2 + jnp.uint32(4)
    x0, x1 = four(x0, x1, _ROT_A)
    x0 = x0 + ks2
    x1 = x1 + ks0 + jnp.uint32(5)
    return x0 ^ x1


def _gumbel_block(t0, n_experts, bt):
    e_iota = jax.lax.broadcasted_iota(jnp.uint32, (n_experts, bt), 0)
    t_iota = jax.lax.broadcasted_iota(jnp.uint32, (n_experts, bt), 1)
    cnt = (jnp.uint32(t0) + t_iota) * jnp.uint32(n_experts) + e_iota
    bits = _threefry_bits(cnt)
    fbits = (bits >> jnp.uint32(9)) | jnp.uint32(0x3F800000)
    flo = jax.lax.bitcast_convert_type(fbits, jnp.float32) - jnp.float32(1.0)
    minval = jnp.float32(1e-20)
    maxval = jnp.float32(1.0)
    u = jnp.maximum(minval, flo * (maxval - minval) + minval)
    return -jnp.log(-jnp.log(u))


def _gate_kernel(xblk_ref, wt_ref, b_ref, o_ref):
    i = pl.program_id(0)
    n_experts = o_ref.shape[0]

    gumbels = _gumbel_block(i * _BT, n_experts, _BT)
    xblk = xblk_ref[...]

    scores = jax.lax.dot_general(
        xblk, wt_ref[...], (((1,), (1,)), ((), ())),
        preferred_element_type=jnp.float32)
    y_t = scores.T + b_ref[...] + gumbels
    m = jnp.max(y_t, axis=0, keepdims=True)
    e_iota = jax.lax.broadcasted_iota(jnp.int32, y_t.shape, 0)
    sel = jnp.min(jnp.where(y_t == m, e_iota, n_experts), axis=0, keepdims=True)
    o_ref[...] = (e_iota == sel).astype(jnp.float32)


@functools.partial(jax.jit, static_argnames=())
def kernel(x, gate_weight, gate_bias):
    n_tokens, d_model = x.shape
    n_experts = gate_weight.shape[0]
    b2 = gate_bias.reshape(n_experts, 1)
    grid = (n_tokens // _BT,)
    return pl.pallas_call(
        _gate_kernel,
        grid=grid,
        in_specs=[
            pl.BlockSpec((_BT, d_model), lambda i: (i, 0)),
            pl.BlockSpec((n_experts, d_model), lambda i: (0, 0)),
            pl.BlockSpec((n_experts, 1), lambda i: (0, 0)),
        ],
        out_specs=pl.BlockSpec((n_experts, _BT), lambda i: (0, i)),
        out_shape=jax.ShapeDtypeStruct((n_experts, n_tokens), x.dtype),
        compiler_params=pltpu.CompilerParams(
            dimension_semantics=("parallel",)),
    )(x, gate_weight, b2).T
